# Initial kernel scaffold; baseline (speedup 1.0000x reference)
import jax
import jax.numpy as jnp
from jax.experimental import pallas as pl


def kernel(x, positions, pos_table):
    raise NotImplementedError("write your pallas kernel here")


# speedup vs baseline: 1.0862x; 1.0862x over previous
import jax
import jax.numpy as jnp
from jax import lax
from jax.experimental import pallas as pl
from jax.experimental.pallas import tpu as pltpu
from jax.experimental.pallas import tpu_sc as plsc

_D = 1024
_LANES = 16
_NC, _NS = 2, 16
_NW = _NC * _NS
_CHUNK = 32


def _pe_body(x_hbm, pos_hbm, tab_hbm, out_hbm, idx_v, xbuf, rowbuf, sem):
    wid = lax.axis_index("s") * _NC + lax.axis_index("c")
    n_rows = pos_hbm.shape[0]
    rows_per_w = n_rows // _NW
    base_w = wid * rows_per_w
    n_chunks = rows_per_w // _CHUNK

    def chunk_body(i, carry):
        base = base_w + i * _CHUNK
        pltpu.sync_copy(pos_hbm.at[pl.ds(base, _CHUNK)], idx_v)
        gather = pltpu.async_copy(tab_hbm.at[idx_v], rowbuf, sem)
        pltpu.sync_copy(x_hbm.at[pl.ds(base, _CHUNK)], xbuf)
        gather.wait()

        def row_body(r, c2):
            for j in range(_D // _LANES):
                off = j * _LANES
                v = rowbuf[r, pl.ds(off, _LANES)]
                plsc.addupdate(xbuf.at[r, pl.ds(off, _LANES)], v)
            return c2

        lax.fori_loop(0, _CHUNK, row_body, 0)
        pltpu.sync_copy(xbuf, out_hbm.at[pl.ds(base, _CHUNK)])
        return carry

    lax.fori_loop(0, n_chunks, chunk_body, 0)


def kernel(x, positions, pos_table):
    b, s, d = x.shape
    n = b * s
    x2 = x.reshape(n, d)
    pos = positions.reshape(n).astype(jnp.int32)
    mesh = plsc.VectorSubcoreMesh(core_axis_name="c", subcore_axis_name="s")
    f = pl.kernel(
        _pe_body,
        mesh=mesh,
        out_type=jax.ShapeDtypeStruct((n, d), jnp.float32),
        scratch_types=[
            pltpu.VMEM((_CHUNK,), jnp.int32),
            pltpu.VMEM((_CHUNK, d), jnp.float32),
            pltpu.VMEM((_CHUNK, d), jnp.float32),
            pltpu.SemaphoreType.DMA,
        ],
    )
    out = f(x2, pos, pos_table)
    return out.reshape(b, s, d)
